# baseline (speedup 1.0000x reference)
import functools

import jax
import jax.numpy as jnp
from jax import lax
from jax.experimental import pallas as pl
from jax.experimental.pallas import tpu as pltpu
from jax.experimental.pallas import tpu_sc as plsc

VOCAB = 1000000
DIM = 16
LABELS = 32
N = 3276800

NC = 2
NS = 16
NW = NC * NS

BPW = N // NW
C = 2048
K = C // 128
NCHUNK = BPW // C


def _sc_gather(idx4d, table):
    mesh = plsc.VectorSubcoreMesh(core_axis_name="c", subcore_axis_name="s")

    @functools.partial(
        pl.kernel,
        mesh=mesh,
        out_type=jax.ShapeDtypeStruct((N, DIM), jnp.float32),
        scratch_types=[
            pltpu.VMEM((K, 128), jnp.int32),
            pltpu.VMEM((C, DIM), jnp.float32),
            pltpu.SemaphoreType.DMA,
        ],
        compiler_params=pltpu.CompilerParams(use_tc_tiling_on_sc=False),
    )
    def k(idx_hbm, table_hbm, out_hbm, idx_v, rows_v, sem):
        wid = lax.axis_index("s") * NC + lax.axis_index("c")

        def body(i, carry):
            pltpu.sync_copy(idx_hbm.at[wid, i], idx_v)
            cps = [
                pltpu.async_copy(
                    table_hbm.at[idx_v.at[j]],
                    rows_v.at[pl.ds(j * 128, 128)],
                    sem,
                )
                for j in range(K)
            ]
            for cp in cps:
                cp.wait()
            pltpu.sync_copy(rows_v, out_hbm.at[pl.ds(wid * BPW + i * C, C)])
            return carry

        lax.fori_loop(0, NCHUNK, body, 0)

    return k(idx4d, table)


def _tc_dense(emb, W, b2d):
    BT = 8192

    def body(emb_ref, w_ref, b_ref, out_ref):
        e = emb_ref[...]
        t = jnp.dot(e, w_ref[...], preferred_element_type=jnp.float32) + b_ref[...]
        m = jnp.max(t, axis=1, keepdims=True)
        s = t - m
        out_ref[...] = s - jnp.log(jnp.sum(jnp.exp(s), axis=1, keepdims=True))

    return pl.pallas_call(
        body,
        grid=(N // BT,),
        in_specs=[
            pl.BlockSpec((BT, DIM), lambda i: (i, 0)),
            pl.BlockSpec((DIM, LABELS), lambda i: (0, 0)),
            pl.BlockSpec((1, LABELS), lambda i: (0, 0)),
        ],
        out_specs=pl.BlockSpec((BT, LABELS), lambda i: (i, 0)),
        out_shape=jax.ShapeDtypeStruct((N, LABELS), jnp.float32),
    )(emb, W, b2d)


def kernel(sentence, table, W, b):
    idx4d = sentence.reshape(NW, NCHUNK, K, 128)
    emb = _sc_gather(idx4d, table)
    scores = _tc_dense(emb, W, b.reshape(1, LABELS))
    return scores, emb


# speedup vs baseline: 2.9976x; 2.9976x over previous
import functools

import jax
import jax.numpy as jnp
from jax import lax
from jax.experimental import pallas as pl
from jax.experimental.pallas import tpu as pltpu
from jax.experimental.pallas import tpu_sc as plsc

VOCAB = 1000000
DIM = 16
LABELS = 32
N = 3276800

NC = 2
NS = 16
NW = NC * NS

BPW = N // NW
C = 2048
K = C // 128
NCHUNK = BPW // C

BT = 8192
M8 = BT // 8
NB = N // BT


def _sc_gather(idx4d, table):
    mesh = plsc.VectorSubcoreMesh(core_axis_name="c", subcore_axis_name="s")

    @functools.partial(
        pl.kernel,
        mesh=mesh,
        out_type=jax.ShapeDtypeStruct((N, DIM), jnp.float32),
        scratch_types=[
            pltpu.VMEM((K, 128), jnp.int32),
            pltpu.VMEM((C, DIM), jnp.float32),
            pltpu.SemaphoreType.DMA,
        ],
        compiler_params=pltpu.CompilerParams(use_tc_tiling_on_sc=False),
    )
    def k(idx_hbm, table_hbm, out_hbm, idx_v, rows_v, sem):
        wid = lax.axis_index("s") * NC + lax.axis_index("c")

        def body(i, carry):
            pltpu.sync_copy(idx_hbm.at[wid, i], idx_v)
            cps = [
                pltpu.async_copy(
                    table_hbm.at[idx_v.at[j]],
                    rows_v.at[pl.ds(j * 128, 128)],
                    sem,
                )
                for j in range(K)
            ]
            for cp in cps:
                cp.wait()
            pltpu.sync_copy(rows_v, out_hbm.at[pl.ds(wid * BPW + i * C, C)])
            return carry

        lax.fori_loop(0, NCHUNK, body, 0)

    return k(idx4d, table)


def _tc_dense(emb8, W, b, eye):
    def body(emb_ref, w_ref, b_ref, eye_ref, scT_ref, embT_ref):
        e8 = emb_ref[...]
        ident = eye_ref[...]
        eT = jnp.concatenate(
            [
                lax.dot_general(
                    ident, e8[:, 16 * g:16 * (g + 1)],
                    (((1,), (1,)), ((), ())),
                    preferred_element_type=jnp.float32)
                for g in range(8)
            ],
            axis=1,
        )
        embT_ref[...] = eT
        t = lax.dot_general(
            w_ref[...], eT, (((0,), (0,)), ((), ())),
            preferred_element_type=jnp.float32) + b_ref[...]
        m = jnp.max(t, axis=0, keepdims=True)
        s = t - m
        scT_ref[...] = s - jnp.log(jnp.sum(jnp.exp(s), axis=0, keepdims=True))

    return pl.pallas_call(
        body,
        grid=(NB,),
        in_specs=[
            pl.BlockSpec((M8, 128), lambda i: (i, 0)),
            pl.BlockSpec((DIM, LABELS), lambda i: (0, 0)),
            pl.BlockSpec((LABELS, 1), lambda i: (0, 0)),
            pl.BlockSpec((DIM, DIM), lambda i: (0, 0)),
        ],
        out_specs=[
            pl.BlockSpec((LABELS, BT), lambda i: (0, i)),
            pl.BlockSpec((DIM, BT), lambda i: (0, i)),
        ],
        out_shape=[
            jax.ShapeDtypeStruct((LABELS, N), jnp.float32),
            jax.ShapeDtypeStruct((DIM, N), jnp.float32),
        ],
    )(emb8, W, b, eye)


def kernel(sentence, table, W, b):
    idx_fed = sentence.reshape(NB, 8, M8).swapaxes(1, 2)
    idx4d = idx_fed.reshape(NW, NCHUNK, K, 128)
    emb_rm = _sc_gather(idx4d, table)
    emb8 = emb_rm.reshape(N // 8, 128)
    eye = jnp.eye(DIM, dtype=jnp.float32)
    scoresT, embT = _tc_dense(emb8, W, b.reshape(LABELS, 1), eye)
    return scoresT.T, embT.T


# speedup vs baseline: 3.0748x; 1.0257x over previous
import functools

import jax
import jax.numpy as jnp
from jax import lax
from jax.experimental import pallas as pl
from jax.experimental.pallas import tpu as pltpu
from jax.experimental.pallas import tpu_sc as plsc

VOCAB = 1000000
DIM = 16
LABELS = 32
N = 3276800

NC = 2
NS = 16
NW = NC * NS

PH = 5
NP = N // PH

BPW = NP // NW
C = 2048
K = C // 128
NCHUNK = BPW // C

BT = 8192
M8 = BT // 8
NBP = NP // BT


def _sc_gather(idx4d, table):
    mesh = plsc.VectorSubcoreMesh(core_axis_name="c", subcore_axis_name="s")

    @functools.partial(
        pl.kernel,
        mesh=mesh,
        out_type=jax.ShapeDtypeStruct((NP, DIM), jnp.float32),
        scratch_types=[
            pltpu.VMEM((K, 128), jnp.int32),
            pltpu.VMEM((C, DIM), jnp.float32),
            pltpu.SemaphoreType.DMA,
        ],
        compiler_params=pltpu.CompilerParams(use_tc_tiling_on_sc=False),
    )
    def k(idx_hbm, table_hbm, out_hbm, idx_v, rows_v, sem):
        wid = lax.axis_index("s") * NC + lax.axis_index("c")

        def body(i, carry):
            pltpu.sync_copy(idx_hbm.at[wid, i], idx_v)
            cps = [
                pltpu.async_copy(
                    table_hbm.at[idx_v.at[j]],
                    rows_v.at[pl.ds(j * 128, 128)],
                    sem,
                )
                for j in range(K)
            ]
            for cp in cps:
                cp.wait()
            pltpu.sync_copy(rows_v, out_hbm.at[pl.ds(wid * BPW + i * C, C)])
            return carry

        lax.fori_loop(0, NCHUNK, body, 0)

    return k(idx4d, table)


def _tc_dense(emb8, W, b, eye, phase, carry):
    def body(emb_ref, w_ref, b_ref, eye_ref, *rest):
        scT_ref, embT_ref = rest[-2], rest[-1]
        e8 = emb_ref[...]
        ident = eye_ref[...]
        eT = jnp.concatenate(
            [
                lax.dot_general(
                    ident, e8[:, 16 * g:16 * (g + 1)],
                    (((1,), (1,)), ((), ())),
                    preferred_element_type=jnp.float32)
                for g in range(8)
            ],
            axis=1,
        )
        embT_ref[...] = eT
        t = lax.dot_general(
            w_ref[...], eT, (((0,), (0,)), ((), ())),
            preferred_element_type=jnp.float32) + b_ref[...]
        m = jnp.max(t, axis=0, keepdims=True)
        s = t - m
        scT_ref[...] = s - jnp.log(jnp.sum(jnp.exp(s), axis=0, keepdims=True))

    in_specs = [
        pl.BlockSpec((M8, 128), lambda i: (i, 0)),
        pl.BlockSpec((DIM, LABELS), lambda i: (0, 0)),
        pl.BlockSpec((LABELS, 1), lambda i: (0, 0)),
        pl.BlockSpec((DIM, DIM), lambda i: (0, 0)),
    ]
    operands = [emb8, W, b, eye]
    kwargs = {}
    if carry is not None:
        in_specs += [
            pl.BlockSpec(memory_space=pl.ANY),
            pl.BlockSpec(memory_space=pl.ANY),
        ]
        operands += [carry[0], carry[1]]
        kwargs["input_output_aliases"] = {4: 0, 5: 1}

    return pl.pallas_call(
        body,
        grid=(NBP,),
        in_specs=in_specs,
        out_specs=[
            pl.BlockSpec((LABELS, BT), lambda i, p=phase: (0, p * NBP + i)),
            pl.BlockSpec((DIM, BT), lambda i, p=phase: (0, p * NBP + i)),
        ],
        out_shape=[
            jax.ShapeDtypeStruct((LABELS, N), jnp.float32),
            jax.ShapeDtypeStruct((DIM, N), jnp.float32),
        ],
        **kwargs,
    )(*operands)


def kernel(sentence, table, W, b):
    eye = jnp.eye(DIM, dtype=jnp.float32)
    b2 = b.reshape(LABELS, 1)
    idx_fed = sentence.reshape(PH, NBP, 8, M8).swapaxes(2, 3)
    emb8s = []
    for p in range(PH):
        idx4d = idx_fed[p].reshape(NW, NCHUNK, K, 128)
        emb_rm = _sc_gather(idx4d, table)
        emb8s.append(emb_rm.reshape(NP // 8, 128))
    carry = None
    for p in range(PH):
        carry = _tc_dense(emb8s[p], W, b2, eye, p, carry)
    scoresT, embT = carry
    return scoresT.T, embT.T


# speedup vs baseline: 3.1951x; 1.0392x over previous
import functools

import jax
import jax.numpy as jnp
from jax import lax
from jax.experimental import pallas as pl
from jax.experimental.pallas import tpu as pltpu
from jax.experimental.pallas import tpu_sc as plsc

VOCAB = 1000000
DIM = 16
LABELS = 32
N = 3276800

NC = 2
NS = 16
NW = NC * NS

PH = 5
NP = N // PH

BPW = NP // NW
C = 2048
K = C // 128
NCHUNK = BPW // C

BT = 8192
MB = BT // 16
NBP = NP // BT


def _sc_gather(idx4d, table):
    mesh = plsc.VectorSubcoreMesh(core_axis_name="c", subcore_axis_name="s")

    @functools.partial(
        pl.kernel,
        mesh=mesh,
        out_type=jax.ShapeDtypeStruct((NP // 16, 128), jnp.int32),
        scratch_types=[
            pltpu.VMEM((K, 128), jnp.int32),
            pltpu.VMEM((C, DIM), jnp.float32),
            pltpu.VMEM((C // 16, 128), jnp.int32),
            pltpu.SemaphoreType.DMA,
        ],
        compiler_params=pltpu.CompilerParams(use_tc_tiling_on_sc=False),
    )
    def k(idx_hbm, table_hbm, out_hbm, idx_v, rows_v, pk_v, sem):
        wid = lax.axis_index("s") * NC + lax.axis_index("c")
        half = jnp.int32(0x7FFF)
        one = jnp.int32(1)
        himask = jnp.int32(-65536)

        def body(i, carry):
            pltpu.sync_copy(idx_hbm.at[wid, i], idx_v)
            cps = [
                pltpu.async_copy(
                    table_hbm.at[idx_v.at[j]],
                    rows_v.at[pl.ds(j * 128, 128)],
                    sem,
                )
                for j in range(K)
            ]
            for cp in cps:
                cp.wait()

            def pbody(r, carry2):
                for u in range(8):
                    x = rows_v[r * 16 + 2 * u]
                    y = rows_v[r * 16 + 2 * u + 1]
                    bx = lax.bitcast_convert_type(x, jnp.int32)
                    by = lax.bitcast_convert_type(y, jnp.int32)
                    rx = bx + half + (lax.shift_right_logical(bx, 16) & one)
                    ry = by + half + (lax.shift_right_logical(by, 16) & one)
                    z = lax.shift_right_logical(rx, 16) | (ry & himask)
                    pk_v[r, pl.ds(u * 16, 16)] = z
                return carry2

            lax.fori_loop(0, C // 16, pbody, 0)
            pltpu.sync_copy(
                pk_v, out_hbm.at[pl.ds((wid * BPW + i * C) // 16, C // 16)])
            return carry

        lax.fori_loop(0, NCHUNK, body, 0)

    return k(idx4d, table)


def _tc_dense(pk, W, b, eye, phase, carry):
    def body(pk_ref, w_ref, b_ref, eye_ref, *rest):
        scT_ref, embT_ref = rest[-2], rest[-1]
        e = pk_ref[...]
        ex = lax.bitcast_convert_type(
            lax.shift_left(e, 16), jnp.float32)
        ey = lax.bitcast_convert_type(
            e & jnp.int32(-65536), jnp.float32)
        ident = eye_ref[...]
        slabs = []
        for g in range(8):
            px = ex[:, 16 * g:16 * (g + 1)]
            py = ey[:, 16 * g:16 * (g + 1)]
            slabs.append(lax.dot_general(
                ident, px, (((1,), (1,)), ((), ())),
                preferred_element_type=jnp.float32))
            slabs.append(lax.dot_general(
                ident, py, (((1,), (1,)), ((), ())),
                preferred_element_type=jnp.float32))
        eT = jnp.concatenate(slabs, axis=1)
        embT_ref[...] = eT
        t = lax.dot_general(
            w_ref[...], eT, (((0,), (0,)), ((), ())),
            preferred_element_type=jnp.float32) + b_ref[...]
        m = jnp.max(t, axis=0, keepdims=True)
        s = t - m
        scT_ref[...] = s - jnp.log(jnp.sum(jnp.exp(s), axis=0, keepdims=True))

    in_specs = [
        pl.BlockSpec((MB, 128), lambda i: (i, 0)),
        pl.BlockSpec((DIM, LABELS), lambda i: (0, 0)),
        pl.BlockSpec((LABELS, 1), lambda i: (0, 0)),
        pl.BlockSpec((DIM, DIM), lambda i: (0, 0)),
    ]
    operands = [pk, W, b, eye]
    kwargs = {}
    if carry is not None:
        in_specs += [
            pl.BlockSpec(memory_space=pl.ANY),
            pl.BlockSpec(memory_space=pl.ANY),
        ]
        operands += [carry[0], carry[1]]
        kwargs["input_output_aliases"] = {4: 0, 5: 1}

    return pl.pallas_call(
        body,
        grid=(NBP,),
        in_specs=in_specs,
        out_specs=[
            pl.BlockSpec((LABELS, BT), lambda i, p=phase: (0, p * NBP + i)),
            pl.BlockSpec((DIM, BT), lambda i, p=phase: (0, p * NBP + i)),
        ],
        out_shape=[
            jax.ShapeDtypeStruct((LABELS, N), jnp.float32),
            jax.ShapeDtypeStruct((DIM, N), jnp.float32),
        ],
        **kwargs,
    )(*operands)


def kernel(sentence, table, W, b):
    eye = jnp.eye(DIM, dtype=jnp.float32)
    b2 = b.reshape(LABELS, 1)
    idx_fed = sentence.reshape(PH, NBP, 16, MB).swapaxes(2, 3)
    pks = []
    for p in range(PH):
        idx4d = idx_fed[p].reshape(NW, NCHUNK, K, 128)
        pks.append(_sc_gather(idx4d, table))
    carry = None
    for p in range(PH):
        carry = _tc_dense(pks[p], W, b2, eye, p, carry)
    scoresT, embT = carry
    return scoresT.T, embT.T


# speedup vs baseline: 3.2271x; 1.0100x over previous
import functools

import jax
import jax.numpy as jnp
from jax import lax
from jax.experimental import pallas as pl
from jax.experimental.pallas import tpu as pltpu
from jax.experimental.pallas import tpu_sc as plsc

VOCAB = 1000000
DIM = 16
LABELS = 32
N = 3276800

NC = 2
NS = 16
NW = NC * NS

PH = 5
NP = N // PH

BPW = NP // NW
C = 2048
K = C // 128
NCHUNK = BPW // C

BT = 16384
MB = BT // 16
NBP = NP // BT


def _sc_gather(idx4d, table):
    mesh = plsc.VectorSubcoreMesh(core_axis_name="c", subcore_axis_name="s")

    @functools.partial(
        pl.kernel,
        mesh=mesh,
        out_type=jax.ShapeDtypeStruct((NP // 16, 128), jnp.int32),
        scratch_types=[
            pltpu.VMEM((K, 128), jnp.int32),
            pltpu.VMEM((C, DIM), jnp.float32),
            pltpu.VMEM((C // 16, 128), jnp.int32),
            pltpu.SemaphoreType.DMA,
        ],
        compiler_params=pltpu.CompilerParams(use_tc_tiling_on_sc=False),
    )
    def k(idx_hbm, table_hbm, out_hbm, idx_v, rows_v, pk_v, sem):
        wid = lax.axis_index("s") * NC + lax.axis_index("c")
        half = jnp.int32(0x7FFF)
        one = jnp.int32(1)
        himask = jnp.int32(-65536)

        def body(i, carry):
            pltpu.sync_copy(idx_hbm.at[wid, i], idx_v)
            cps = [
                pltpu.async_copy(
                    table_hbm.at[idx_v.at[j]],
                    rows_v.at[pl.ds(j * 128, 128)],
                    sem,
                )
                for j in range(K)
            ]
            for cp in cps:
                cp.wait()

            def pbody(r, carry2):
                for u in range(8):
                    x = rows_v[r * 16 + 2 * u]
                    y = rows_v[r * 16 + 2 * u + 1]
                    bx = lax.bitcast_convert_type(x, jnp.int32)
                    by = lax.bitcast_convert_type(y, jnp.int32)
                    rx = bx + half + (lax.shift_right_logical(bx, 16) & one)
                    ry = by + half + (lax.shift_right_logical(by, 16) & one)
                    z = lax.shift_right_logical(rx, 16) | (ry & himask)
                    pk_v[r, pl.ds(u * 16, 16)] = z
                return carry2

            lax.fori_loop(0, C // 16, pbody, 0)
            pltpu.sync_copy(
                pk_v, out_hbm.at[pl.ds((wid * BPW + i * C) // 16, C // 16)])
            return carry

        lax.fori_loop(0, NCHUNK, body, 0)

    return k(idx4d, table)


def _tc_dense(pk, W, b, eye, phase, carry):
    def body(pk_ref, w_ref, b_ref, eye_ref, *rest):
        scT_ref, embT_ref = rest[-2], rest[-1]
        e = pk_ref[...]
        ex = lax.bitcast_convert_type(
            lax.shift_left(e, 16), jnp.float32)
        ey = lax.bitcast_convert_type(
            e & jnp.int32(-65536), jnp.float32)
        ident = eye_ref[...]
        slabs = []
        for g in range(8):
            px = ex[:, 16 * g:16 * (g + 1)]
            py = ey[:, 16 * g:16 * (g + 1)]
            slabs.append(lax.dot_general(
                ident, px, (((1,), (1,)), ((), ())),
                preferred_element_type=jnp.float32))
            slabs.append(lax.dot_general(
                ident, py, (((1,), (1,)), ((), ())),
                preferred_element_type=jnp.float32))
        eT = jnp.concatenate(slabs, axis=1)
        embT_ref[...] = eT
        t = lax.dot_general(
            w_ref[...], eT, (((0,), (0,)), ((), ())),
            preferred_element_type=jnp.float32) + b_ref[...]
        m = jnp.max(t, axis=0, keepdims=True)
        s = t - m
        scT_ref[...] = s - jnp.log(jnp.sum(jnp.exp(s), axis=0, keepdims=True))

    in_specs = [
        pl.BlockSpec((MB, 128), lambda i: (i, 0)),
        pl.BlockSpec((DIM, LABELS), lambda i: (0, 0)),
        pl.BlockSpec((LABELS, 1), lambda i: (0, 0)),
        pl.BlockSpec((DIM, DIM), lambda i: (0, 0)),
    ]
    operands = [pk, W, b, eye]
    kwargs = {}
    if carry is not None:
        in_specs += [
            pl.BlockSpec(memory_space=pl.ANY),
            pl.BlockSpec(memory_space=pl.ANY),
        ]
        operands += [carry[0], carry[1]]
        kwargs["input_output_aliases"] = {4: 0, 5: 1}

    return pl.pallas_call(
        body,
        grid=(NBP,),
        in_specs=in_specs,
        out_specs=[
            pl.BlockSpec((LABELS, BT), lambda i, p=phase: (0, p * NBP + i)),
            pl.BlockSpec((DIM, BT), lambda i, p=phase: (0, p * NBP + i)),
        ],
        out_shape=[
            jax.ShapeDtypeStruct((LABELS, N), jnp.float32),
            jax.ShapeDtypeStruct((DIM, N), jnp.float32),
        ],
        **kwargs,
    )(*operands)


def kernel(sentence, table, W, b):
    eye = jnp.eye(DIM, dtype=jnp.float32)
    b2 = b.reshape(LABELS, 1)
    idx_fed = sentence.reshape(PH, NBP, 16, MB).swapaxes(2, 3)
    pks = []
    for p in range(PH):
        idx4d = idx_fed[p].reshape(NW, NCHUNK, K, 128)
        pks.append(_sc_gather(idx4d, table))
    carry = None
    for p in range(PH):
        carry = _tc_dense(pks[p], W, b2, eye, p, carry)
    scoresT, embT = carry
    return scoresT.T, embT.T
